# baseline (speedup 1.0000x reference)
import functools
import math

import jax
import jax.numpy as jnp
from jax import lax
from jax.experimental import pallas as pl
from jax.experimental.pallas import tpu as pltpu
from jax.experimental.pallas import tpu_sc as plsc

N_SENSORS = 4760
K = 16
HID = 64
LPROJ = 128
POSD = 96
FED = 32
OUTC = 2
NH = 4
HD = LPROJ // NH
NTOK = 6
B = 8

N_PAD = 4864
S_BLK = 256
GATHER_WIN = 128
N_IDX_PAD = 4864
HALF_A = 2560


def _pre_body(pos_ref, fid_ref, femb_ref, tfemb_ref, lat_ref,
              x01t_ref, mkt_ref,
              wpf_p_ref, wpf_f_ref, bn_ref, wv_ref,
              wql_p_ref, wql_f_ref, bql_ref,
              wqg_p_ref, wqg_f_ref, bqg_ref,
              wlat_ref, blat_ref, wlf_ref, blf_ref,
              wk_ref, bk_ref, wv2_ref, bv2_ref,
              table_out, ql_out, a_out, qg_out, msk_out, kg_out, vg_out):
    n = pos_ref.shape[0]
    pos = pos_ref[...]
    fid = fid_ref[...]
    oh = (fid == lax.broadcasted_iota(jnp.int32, (1, 6), 1)).astype(jnp.float32)
    fpart = jnp.dot(oh, femb_ref[...], preferred_element_type=jnp.float32)

    pf = (jnp.dot(pos, wpf_p_ref[...], preferred_element_type=jnp.float32)
          + jnp.dot(fpart, wpf_f_ref[...], preferred_element_type=jnp.float32)
          + bn_ref[...])
    table_out[:, 0:HID] = pf
    table_out[:, HID:HID + 16] = x01t_ref[...]
    table_out[:, HID + 16:HID + 24] = mkt_ref[...]
    table_out[:, HID + 24:] = jnp.zeros((n, LPROJ - HID - 24), jnp.float32)

    ql = (jnp.dot(pos, wql_p_ref[...], preferred_element_type=jnp.float32)
          + jnp.dot(fpart, wql_f_ref[...], preferred_element_type=jnp.float32)
          + bql_ref[...]) * (HID ** -0.5)
    ql_out[0:n] = ql
    ql_out[n:] = jnp.zeros((N_PAD - n, HID), jnp.float32)

    a = lax.dot_general(ql, wv_ref[...], (((1,), (1,)), ((), ())),
                        preferred_element_type=jnp.float32)
    a_out[0:n] = a
    a_out[n:] = jnp.zeros((N_PAD - n, 2), jnp.float32)

    qg = (jnp.dot(pos, wqg_p_ref[...], preferred_element_type=jnp.float32)
          + jnp.dot(fpart, wqg_f_ref[...], preferred_element_type=jnp.float32)
          + bqg_ref[...]) * (HD ** -0.5)
    qg_out[0:n] = qg
    qg_out[n:] = jnp.zeros((N_PAD - n, LPROJ), jnp.float32)

    msk_out[0:n] = mkt_ref[...]
    msk_out[n:] = jnp.zeros((N_PAD - n, 8), jnp.float32)

    bias = jnp.dot(tfemb_ref[...], wlf_ref[...],
                   preferred_element_type=jnp.float32) + blf_ref[...]
    kv = jnp.dot(lat_ref[...], wlat_ref[...],
                 preferred_element_type=jnp.float32) + blat_ref[...] + bias
    kg_out[...] = jnp.dot(kv, wk_ref[...],
                          preferred_element_type=jnp.float32) + bk_ref[...]
    vg_out[...] = jnp.dot(kv, wv2_ref[...],
                          preferred_element_type=jnp.float32) + bv2_ref[...]


def _precompute(pos_embed, face_idx, face_emb, tf_emb_tiled, latent2d,
                x01t, mkt,
                wpf_p, wpf_f, bn, wv01,
                wql_p, wql_f, bql, wqg_p, wqg_f, bqg,
                wlat, blat, wlf, blf, wk, bk, wv2, bv2):
    n = pos_embed.shape[0]
    out_shapes = (
        jax.ShapeDtypeStruct((n, LPROJ), jnp.float32),
        jax.ShapeDtypeStruct((N_PAD, HID), jnp.float32),
        jax.ShapeDtypeStruct((N_PAD, 2), jnp.float32),
        jax.ShapeDtypeStruct((N_PAD, LPROJ), jnp.float32),
        jax.ShapeDtypeStruct((N_PAD, 8), jnp.float32),
        jax.ShapeDtypeStruct((B * NTOK, LPROJ), jnp.float32),
        jax.ShapeDtypeStruct((B * NTOK, LPROJ), jnp.float32),
    )
    return pl.pallas_call(_pre_body, out_shape=out_shapes)(
        pos_embed, face_idx, face_emb, tf_emb_tiled, latent2d, x01t, mkt,
        wpf_p, wpf_f, bn, wv01, wql_p, wql_f, bql, wqg_p, wqg_f, bqg,
        wlat, blat, wlf, blf, wk, bk, wv2, bv2)


def _sc_gather(table, idx_flat):
    ni = idx_flat.shape[0]
    idx2 = idx_flat.reshape(1, ni)
    mesh = plsc.VectorSubcoreMesh(core_axis_name="core",
                                  subcore_axis_name="subcore")

    @functools.partial(
        pl.kernel,
        out_type=jax.ShapeDtypeStruct((ni, LPROJ), jnp.float32),
        mesh=mesh)
    def kern(t_hbm, i_hbm, o_hbm):
        def body(i_vmem, o_vmem):
            pltpu.sync_copy(t_hbm.at[i_vmem.at[0]], o_vmem)

        pltpu.emit_pipeline(
            body,
            grid=(ni // GATHER_WIN,),
            in_specs=[pl.BlockSpec((1, GATHER_WIN), index_map=lambda i: (0, i))],
            out_specs=[pl.BlockSpec((GATHER_WIN, LPROJ),
                                    index_map=lambda i: (i, 0))],
            core_axis_name=("core", "subcore"),
            dimension_semantics=(pltpu.PARALLEL,),
        )(i_hbm, o_hbm)

    return kern(table, idx2)


def _main_body(g_ref, ql_ref, a_ref, qg_ref, msk_ref,
               kmat_ref, vmat_ref, wv_ref, wgo_ref, bgo_ref,
               lng_l_ref, lnb_l_ref, lng_g_ref, lnb_g_ref,
               wm1_l_ref, wm1_g_ref, bm1_ref, wm2_ref, bm2_ref,
               out_ref):
    s = g_ref.shape[0]
    ql = ql_ref[...]
    a0 = a_ref[:, 0:1]
    a1 = a_ref[:, 1:2]
    qg = qg_ref[...]
    g = g_ref[...]

    bf = []
    v01s = []
    logit = []
    for k in range(K):
        ch = g[:, k * LPROJ:(k + 1) * LPROJ]
        bfk = ch[:, 0:HID]
        bf.append(bfk)
        bd = jnp.sum(bfk * ql, axis=1, keepdims=True)
        v01 = ch[:, 64:80]
        mk = ch[:, 80:88]
        v01s.append(v01)
        lk = bd + v01[:, 0:8] * a0 + v01[:, 8:16] * a1
        logit.append(jnp.where(mk != 0.0, -10000.0, lk))

    mx = logit[0]
    for k in range(1, K):
        mx = jnp.maximum(mx, logit[k])
    es = [jnp.exp(logit[k] - mx) for k in range(K)]
    z = es[0]
    for k in range(1, K):
        z = z + es[k]
    rz = 1.0 / z

    wv0 = es[0] * v01s[0][:, 0:8]
    wv1 = es[0] * v01s[0][:, 8:16]
    for k in range(1, K):
        wv0 = wv0 + es[k] * v01s[k][:, 0:8]
        wv1 = wv1 + es[k] * v01s[k][:, 8:16]
    wv0 = wv0 * rz
    wv1 = wv1 * rz

    wv = wv_ref[...]
    wgo = wgo_ref[...]
    bgo = bgo_ref[...]
    lng_l = lng_l_ref[...]
    lnb_l = lnb_l_ref[...]
    lng_g = lng_g_ref[...]
    lnb_g = lnb_g_ref[...]
    wm1_l = wm1_l_ref[...]
    wm1_g = wm1_g_ref[...]
    bm1 = bm1_ref[...]
    wm2 = wm2_ref[...]
    bm2 = bm2_ref[...]
    col8 = lax.broadcasted_iota(jnp.int32, (1, 8), 1)
    inv_d = 1.0 / (HID + LPROJ)
    inv_sqrt2 = 1.0 / math.sqrt(2.0)

    for b in range(B):
        ebk = es[0][:, b:b + 1]
        wb = ebk * bf[0]
        for k in range(1, K):
            ebk = es[k][:, b:b + 1]
            wb = wb + ebk * bf[k]
        rzb = rz[:, b:b + 1]
        local = (wb * rzb
                 + wv0[:, b:b + 1] * wv[0:1, :]
                 + wv1[:, b:b + 1] * wv[1:2, :])

        kb = kmat_ref[b * LPROJ:(b + 1) * LPROJ, :]
        lg = jnp.dot(qg, kb, preferred_element_type=jnp.float32)
        ctx = jnp.zeros((s, LPROJ), dtype=jnp.float32)
        for h in range(NH):
            sl = lg[:, 8 * h:8 * h + 8]
            sl = jnp.where(col8 < NTOK, sl, -1e30)
            mh = jnp.max(sl, axis=1, keepdims=True)
            eh = jnp.exp(sl - mh)
            attn_h = eh / jnp.sum(eh, axis=1, keepdims=True)
            vb = vmat_ref[b * 32 + 8 * h:b * 32 + 8 * h + 8, :]
            ctx = ctx + jnp.dot(attn_h, vb, preferred_element_type=jnp.float32)
        gf = jnp.dot(ctx, wgo, preferred_element_type=jnp.float32) + bgo

        s1 = jnp.sum(local, axis=1, keepdims=True) + jnp.sum(gf, axis=1, keepdims=True)
        mu = s1 * inv_d
        s2 = (jnp.sum(local * local, axis=1, keepdims=True)
              + jnp.sum(gf * gf, axis=1, keepdims=True))
        var = s2 * inv_d - mu * mu
        rstd = lax.rsqrt(var + 1e-5)
        xl = (local - mu) * rstd * lng_l + lnb_l
        xg = (gf - mu) * rstd * lng_g + lnb_g
        h1 = (jnp.dot(xl, wm1_l, preferred_element_type=jnp.float32)
              + jnp.dot(xg, wm1_g, preferred_element_type=jnp.float32) + bm1)
        h1 = 0.5 * h1 * (1.0 + lax.erf(h1 * inv_sqrt2))
        preds = jnp.dot(h1, wm2, preferred_element_type=jnp.float32) + bm2
        mb = msk_ref[:, b:b + 1]
        out_ref[b] = preds * (mb != 0.0).astype(jnp.float32)


def _main(g, ql, a2, qg, msk, kmat, vmat, wv8, wgo, bgo,
          lng_l, lnb_l, lng_g, lnb_g, wm1_l, wm1_g, bm1, wm2, bm2,
          off_blk=0):
    n_rows = g.shape[0]
    grid = (n_rows // S_BLK,)
    full = lambda shp: pl.BlockSpec(shp, lambda i: tuple(0 for _ in shp))
    gblk = pl.BlockSpec((S_BLK, K * LPROJ), lambda i: (i, 0))
    blk = lambda cols: pl.BlockSpec((S_BLK, cols), lambda i: (i + off_blk, 0))
    in_specs = [
        gblk, blk(HID), blk(2), blk(LPROJ), blk(8),
        full(kmat.shape), full(vmat.shape), full(wv8.shape), full(wgo.shape),
        full(bgo.shape), full(lng_l.shape), full(lnb_l.shape),
        full(lng_g.shape), full(lnb_g.shape), full(wm1_l.shape),
        full(wm1_g.shape), full(bm1.shape), full(wm2.shape), full(bm2.shape),
    ]
    out_spec = pl.BlockSpec((B, S_BLK, OUTC), lambda i: (0, i, 0))
    return pl.pallas_call(
        _main_body,
        grid=grid,
        in_specs=in_specs,
        out_specs=out_spec,
        out_shape=jax.ShapeDtypeStruct((B, n_rows, OUTC), jnp.float32),
    )(g, ql, a2, qg, msk, kmat, vmat, wv8, wgo, bgo,
      lng_l, lnb_l, lng_g, lnb_g, wm1_l, wm1_g, bm1, wm2, bm2)


def kernel(x_flat, latent_seq, mask, pos_embed, knn_indices, face_ids,
           token_face_ids_map, face_emb, W_nbr, b_nbr, W_ql, b_ql,
           W_lat, b_lat, W_lf, b_lf, W_qg, b_qg, W_k, b_k, W_v, b_v,
           W_go, b_go, ln_g, ln_b, W_m1, b_m1, W_m2, b_m2):
    n = x_flat.shape[1]
    f32 = jnp.float32

    wv01 = W_nbr[:2]
    wv8 = jnp.concatenate([wv01, jnp.zeros((6, HID), f32)], axis=0)
    wpf_p, wpf_f = W_nbr[2:2 + POSD], W_nbr[2 + POSD:]
    wql_p, wql_f = W_ql[:POSD], W_ql[POSD:]
    wqg_p, wqg_f = W_qg[:POSD], W_qg[POSD:]
    row = lambda v: v.reshape(1, -1).astype(f32)
    face_idx = face_ids.astype(jnp.int32).reshape(n, 1)
    tf_emb = face_emb[token_face_ids_map]
    tf_emb_tiled = jnp.tile(tf_emb, (B, 1))
    latent2d = latent_seq.reshape(B * NTOK, -1)
    x01t = x_flat.astype(f32).transpose(1, 2, 0).reshape(n, 16)
    mkt = mask.T.astype(f32)

    table, ql, a2, qg, msk2d, kg, vg = _precompute(
        pos_embed.astype(f32), face_idx, face_emb, tf_emb_tiled, latent2d,
        x01t, mkt,
        wpf_p, wpf_f, row(b_nbr), wv01,
        wql_p, wql_f, row(b_ql), wqg_p, wqg_f, row(b_qg),
        W_lat, row(b_lat), W_lf, row(b_lf), W_k, row(b_k), W_v, row(b_v))

    knn_pad = jnp.zeros((N_IDX_PAD, K), jnp.int32).at[:n].set(
        knn_indices.astype(jnp.int32))
    idx_flat = knn_pad.reshape(-1)
    ga = _sc_gather(table, idx_flat[:HALF_A * K]).reshape(HALF_A, K * LPROJ)
    gb = _sc_gather(table, idx_flat[HALF_A * K:]).reshape(
        N_IDX_PAD - HALF_A, K * LPROJ)

    k3 = kg.reshape(B, NTOK, NH, HD)
    v3 = vg.reshape(B, NTOK, NH, HD)
    kmat = jnp.zeros((B, LPROJ, 32), f32)
    vmat = jnp.zeros((B, NH, 8, LPROJ), f32)
    for h in range(NH):
        kmat = kmat.at[:, HD * h:HD * (h + 1), 8 * h:8 * h + NTOK].set(
            k3[:, :, h, :].transpose(0, 2, 1))
        vmat = vmat.at[:, h, :NTOK, HD * h:HD * (h + 1)].set(v3[:, :, h, :])
    kmat = kmat.reshape(B * LPROJ, 32)
    vmat = vmat.reshape(B * NH * 8, LPROJ)

    consts = (kmat, vmat, wv8, W_go, row(b_go),
              row(ln_g[:HID]), row(ln_b[:HID]),
              row(ln_g[HID:]), row(ln_b[HID:]),
              W_m1[:HID], W_m1[HID:], row(b_m1), W_m2, row(b_m2))
    out_a = _main(ga, ql, a2, qg, msk2d, *consts, off_blk=0)
    out_b = _main(gb, ql, a2, qg, msk2d, *consts, off_blk=HALF_A // S_BLK)
    out = jnp.concatenate([out_a, out_b], axis=1)
    return out[:, :n, :].astype(x_flat.dtype)


# speedup vs baseline: 1.0166x; 1.0166x over previous
import functools
import math

import jax
import jax.numpy as jnp
from jax import lax
from jax.experimental import pallas as pl
from jax.experimental.pallas import tpu as pltpu
from jax.experimental.pallas import tpu_sc as plsc

N_SENSORS = 4760
K = 16
HID = 64
LPROJ = 128
POSD = 96
FED = 32
OUTC = 2
NH = 4
HD = LPROJ // NH
NTOK = 6
B = 8

N_PAD = 4864
S_BLK = 256
GATHER_WIN = 128
N_IDX_PAD = 4864


def _pre_body(pos_ref, fid_ref, femb_ref, tfemb_ref, lat_ref,
              x01t_ref, mkt_ref,
              wpf_p_ref, wpf_f_ref, bn_ref, wv_ref,
              wql_p_ref, wql_f_ref, bql_ref,
              wqg_p_ref, wqg_f_ref, bqg_ref,
              wlat_ref, blat_ref, wlf_ref, blf_ref,
              wk_ref, bk_ref, wv2_ref, bv2_ref,
              table_out, ql_out, a_out, qg_out, msk_out, kg_out, vg_out):
    n = pos_ref.shape[0]
    pos = pos_ref[...]
    fid = fid_ref[...]
    oh = (fid == lax.broadcasted_iota(jnp.int32, (1, 6), 1)).astype(jnp.float32)
    fpart = jnp.dot(oh, femb_ref[...], preferred_element_type=jnp.float32)

    pf = (jnp.dot(pos, wpf_p_ref[...], preferred_element_type=jnp.float32)
          + jnp.dot(fpart, wpf_f_ref[...], preferred_element_type=jnp.float32)
          + bn_ref[...])
    table_out[:, 0:HID] = pf
    table_out[:, HID:HID + 16] = x01t_ref[...]
    table_out[:, HID + 16:HID + 24] = mkt_ref[...]
    table_out[:, HID + 24:] = jnp.zeros((n, LPROJ - HID - 24), jnp.float32)

    ql = (jnp.dot(pos, wql_p_ref[...], preferred_element_type=jnp.float32)
          + jnp.dot(fpart, wql_f_ref[...], preferred_element_type=jnp.float32)
          + bql_ref[...]) * (HID ** -0.5)
    ql_out[0:n] = ql
    ql_out[n:] = jnp.zeros((N_PAD - n, HID), jnp.float32)

    a = lax.dot_general(ql, wv_ref[...], (((1,), (1,)), ((), ())),
                        preferred_element_type=jnp.float32)
    a_out[0:n] = a
    a_out[n:] = jnp.zeros((N_PAD - n, 2), jnp.float32)

    qg = (jnp.dot(pos, wqg_p_ref[...], preferred_element_type=jnp.float32)
          + jnp.dot(fpart, wqg_f_ref[...], preferred_element_type=jnp.float32)
          + bqg_ref[...]) * (HD ** -0.5)
    qg_out[0:n] = qg
    qg_out[n:] = jnp.zeros((N_PAD - n, LPROJ), jnp.float32)

    msk_out[0:n] = mkt_ref[...]
    msk_out[n:] = jnp.zeros((N_PAD - n, 8), jnp.float32)

    bias = jnp.dot(tfemb_ref[...], wlf_ref[...],
                   preferred_element_type=jnp.float32) + blf_ref[...]
    kv = jnp.dot(lat_ref[...], wlat_ref[...],
                 preferred_element_type=jnp.float32) + blat_ref[...] + bias
    kg_out[...] = jnp.dot(kv, wk_ref[...],
                          preferred_element_type=jnp.float32) + bk_ref[...]
    vg_out[...] = jnp.dot(kv, wv2_ref[...],
                          preferred_element_type=jnp.float32) + bv2_ref[...]


def _precompute(pos_embed, face_idx, face_emb, tf_emb_tiled, latent2d,
                x01t, mkt,
                wpf_p, wpf_f, bn, wv01,
                wql_p, wql_f, bql, wqg_p, wqg_f, bqg,
                wlat, blat, wlf, blf, wk, bk, wv2, bv2):
    n = pos_embed.shape[0]
    out_shapes = (
        jax.ShapeDtypeStruct((n, LPROJ), jnp.float32),
        jax.ShapeDtypeStruct((N_PAD, HID), jnp.float32),
        jax.ShapeDtypeStruct((N_PAD, 2), jnp.float32),
        jax.ShapeDtypeStruct((N_PAD, LPROJ), jnp.float32),
        jax.ShapeDtypeStruct((N_PAD, 8), jnp.float32),
        jax.ShapeDtypeStruct((B * NTOK, LPROJ), jnp.float32),
        jax.ShapeDtypeStruct((B * NTOK, LPROJ), jnp.float32),
    )
    return pl.pallas_call(_pre_body, out_shape=out_shapes)(
        pos_embed, face_idx, face_emb, tf_emb_tiled, latent2d, x01t, mkt,
        wpf_p, wpf_f, bn, wv01, wql_p, wql_f, bql, wqg_p, wqg_f, bqg,
        wlat, blat, wlf, blf, wk, bk, wv2, bv2)


def _sc_gather(table, idx_flat):
    ni = idx_flat.shape[0]
    idx2 = idx_flat.reshape(1, ni)
    mesh = plsc.VectorSubcoreMesh(core_axis_name="core",
                                  subcore_axis_name="subcore")

    @functools.partial(
        pl.kernel,
        out_type=jax.ShapeDtypeStruct((ni, LPROJ), jnp.float32),
        mesh=mesh)
    def kern(t_hbm, i_hbm, o_hbm):
        def body(i_vmem, o_vmem):
            pltpu.sync_copy(t_hbm.at[i_vmem.at[0]], o_vmem)

        pltpu.emit_pipeline(
            body,
            grid=(ni // GATHER_WIN,),
            in_specs=[pl.BlockSpec((1, GATHER_WIN), index_map=lambda i: (0, i))],
            out_specs=[pl.BlockSpec((GATHER_WIN, LPROJ),
                                    index_map=lambda i: (i, 0))],
            core_axis_name=("core", "subcore"),
            dimension_semantics=(pltpu.PARALLEL,),
        )(i_hbm, o_hbm)

    return kern(table, idx2)


def _main_body(g_ref, ql_ref, a_ref, qg_ref, msk_ref,
               kmat_ref, vmat_ref, wv_ref, wgo_ref, bgo_ref,
               lng_l_ref, lnb_l_ref, lng_g_ref, lnb_g_ref,
               wm1_l_ref, wm1_g_ref, bm1_ref, wm2_ref, bm2_ref,
               out_ref):
    s = g_ref.shape[0]
    ql = ql_ref[...]
    a0 = a_ref[:, 0:1]
    a1 = a_ref[:, 1:2]
    qg = qg_ref[...]
    g = g_ref[...]

    bf = []
    v01s = []
    logit = []
    for k in range(K):
        ch = g[:, k * LPROJ:(k + 1) * LPROJ]
        bfk = ch[:, 0:HID]
        bf.append(bfk)
        bd = jnp.sum(bfk * ql, axis=1, keepdims=True)
        v01 = ch[:, 64:80]
        mk = ch[:, 80:88]
        v01s.append(v01)
        lk = bd + v01[:, 0:8] * a0 + v01[:, 8:16] * a1
        logit.append(jnp.where(mk != 0.0, -10000.0, lk))

    mx = logit[0]
    for k in range(1, K):
        mx = jnp.maximum(mx, logit[k])
    es = [jnp.exp(logit[k] - mx) for k in range(K)]
    z = es[0]
    for k in range(1, K):
        z = z + es[k]
    rz = 1.0 / z

    wv0 = es[0] * v01s[0][:, 0:8]
    wv1 = es[0] * v01s[0][:, 8:16]
    for k in range(1, K):
        wv0 = wv0 + es[k] * v01s[k][:, 0:8]
        wv1 = wv1 + es[k] * v01s[k][:, 8:16]
    wv0 = wv0 * rz
    wv1 = wv1 * rz

    wv = wv_ref[...]
    wgo = wgo_ref[...]
    bgo = bgo_ref[...]
    lng_l = lng_l_ref[...]
    lnb_l = lnb_l_ref[...]
    lng_g = lng_g_ref[...]
    lnb_g = lnb_g_ref[...]
    wm1_l = wm1_l_ref[...]
    wm1_g = wm1_g_ref[...]
    bm1 = bm1_ref[...]
    wm2 = wm2_ref[...]
    bm2 = bm2_ref[...]
    col8 = lax.broadcasted_iota(jnp.int32, (1, 8), 1)
    inv_d = 1.0 / (HID + LPROJ)
    inv_sqrt2 = 1.0 / math.sqrt(2.0)

    for b in range(B):
        ebk = es[0][:, b:b + 1]
        wb = ebk * bf[0]
        for k in range(1, K):
            ebk = es[k][:, b:b + 1]
            wb = wb + ebk * bf[k]
        rzb = rz[:, b:b + 1]
        local = (wb * rzb
                 + wv0[:, b:b + 1] * wv[0:1, :]
                 + wv1[:, b:b + 1] * wv[1:2, :])

        kb = kmat_ref[b * LPROJ:(b + 1) * LPROJ, :]
        lg = jnp.dot(qg, kb, preferred_element_type=jnp.float32)
        ctx = jnp.zeros((s, LPROJ), dtype=jnp.float32)
        for h in range(NH):
            sl = lg[:, 8 * h:8 * h + 8]
            sl = jnp.where(col8 < NTOK, sl, -1e30)
            mh = jnp.max(sl, axis=1, keepdims=True)
            eh = jnp.exp(sl - mh)
            attn_h = eh / jnp.sum(eh, axis=1, keepdims=True)
            vb = vmat_ref[b * 32 + 8 * h:b * 32 + 8 * h + 8, :]
            ctx = ctx + jnp.dot(attn_h, vb, preferred_element_type=jnp.float32)
        gf = jnp.dot(ctx, wgo, preferred_element_type=jnp.float32) + bgo

        s1 = jnp.sum(local, axis=1, keepdims=True) + jnp.sum(gf, axis=1, keepdims=True)
        mu = s1 * inv_d
        s2 = (jnp.sum(local * local, axis=1, keepdims=True)
              + jnp.sum(gf * gf, axis=1, keepdims=True))
        var = s2 * inv_d - mu * mu
        rstd = lax.rsqrt(var + 1e-5)
        xl = (local - mu) * rstd * lng_l + lnb_l
        xg = (gf - mu) * rstd * lng_g + lnb_g
        h1 = (jnp.dot(xl, wm1_l, preferred_element_type=jnp.float32)
              + jnp.dot(xg, wm1_g, preferred_element_type=jnp.float32) + bm1)
        h1 = 0.5 * h1 * (1.0 + lax.erf(h1 * inv_sqrt2))
        preds = jnp.dot(h1, wm2, preferred_element_type=jnp.float32) + bm2
        mb = msk_ref[:, b:b + 1]
        out_ref[b] = preds * (mb != 0.0).astype(jnp.float32)


def _main(g, ql, a2, qg, msk, kmat, vmat, wv8, wgo, bgo,
          lng_l, lnb_l, lng_g, lnb_g, wm1_l, wm1_g, bm1, wm2, bm2,
          off_blk=0):
    n_rows = g.shape[0]
    grid = (n_rows // S_BLK,)
    full = lambda shp: pl.BlockSpec(shp, lambda i: tuple(0 for _ in shp))
    gblk = pl.BlockSpec((S_BLK, K * LPROJ), lambda i: (i, 0))
    blk = lambda cols: pl.BlockSpec((S_BLK, cols), lambda i: (i + off_blk, 0))
    in_specs = [
        gblk, blk(HID), blk(2), blk(LPROJ), blk(8),
        full(kmat.shape), full(vmat.shape), full(wv8.shape), full(wgo.shape),
        full(bgo.shape), full(lng_l.shape), full(lnb_l.shape),
        full(lng_g.shape), full(lnb_g.shape), full(wm1_l.shape),
        full(wm1_g.shape), full(bm1.shape), full(wm2.shape), full(bm2.shape),
    ]
    out_spec = pl.BlockSpec((B, S_BLK, OUTC), lambda i: (0, i, 0))
    return pl.pallas_call(
        _main_body,
        grid=grid,
        in_specs=in_specs,
        out_specs=out_spec,
        out_shape=jax.ShapeDtypeStruct((B, n_rows, OUTC), jnp.float32),
    )(g, ql, a2, qg, msk, kmat, vmat, wv8, wgo, bgo,
      lng_l, lnb_l, lng_g, lnb_g, wm1_l, wm1_g, bm1, wm2, bm2)


def kernel(x_flat, latent_seq, mask, pos_embed, knn_indices, face_ids,
           token_face_ids_map, face_emb, W_nbr, b_nbr, W_ql, b_ql,
           W_lat, b_lat, W_lf, b_lf, W_qg, b_qg, W_k, b_k, W_v, b_v,
           W_go, b_go, ln_g, ln_b, W_m1, b_m1, W_m2, b_m2):
    n = x_flat.shape[1]
    f32 = jnp.float32

    wv01 = W_nbr[:2]
    wv8 = jnp.concatenate([wv01, jnp.zeros((6, HID), f32)], axis=0)
    wpf_p, wpf_f = W_nbr[2:2 + POSD], W_nbr[2 + POSD:]
    wql_p, wql_f = W_ql[:POSD], W_ql[POSD:]
    wqg_p, wqg_f = W_qg[:POSD], W_qg[POSD:]
    row = lambda v: v.reshape(1, -1).astype(f32)
    face_idx = face_ids.astype(jnp.int32).reshape(n, 1)
    tf_emb = face_emb[token_face_ids_map]
    tf_emb_tiled = jnp.tile(tf_emb, (B, 1))
    latent2d = latent_seq.reshape(B * NTOK, -1)
    x01t = x_flat.astype(f32).transpose(1, 2, 0).reshape(n, 16)
    mkt = mask.T.astype(f32)

    table, ql, a2, qg, msk2d, kg, vg = _precompute(
        pos_embed.astype(f32), face_idx, face_emb, tf_emb_tiled, latent2d,
        x01t, mkt,
        wpf_p, wpf_f, row(b_nbr), wv01,
        wql_p, wql_f, row(b_ql), wqg_p, wqg_f, row(b_qg),
        W_lat, row(b_lat), W_lf, row(b_lf), W_k, row(b_k), W_v, row(b_v))

    knn_pad = jnp.zeros((N_IDX_PAD, K), jnp.int32).at[:n].set(
        knn_indices.astype(jnp.int32))
    idx_flat = knn_pad.reshape(-1)
    g = _sc_gather(table, idx_flat).reshape(N_IDX_PAD, K * LPROJ)

    k3 = kg.reshape(B, NTOK, NH, HD)
    v3 = vg.reshape(B, NTOK, NH, HD)
    kmat = jnp.zeros((B, LPROJ, 32), f32)
    vmat = jnp.zeros((B, NH, 8, LPROJ), f32)
    for h in range(NH):
        kmat = kmat.at[:, HD * h:HD * (h + 1), 8 * h:8 * h + NTOK].set(
            k3[:, :, h, :].transpose(0, 2, 1))
        vmat = vmat.at[:, h, :NTOK, HD * h:HD * (h + 1)].set(v3[:, :, h, :])
    kmat = kmat.reshape(B * LPROJ, 32)
    vmat = vmat.reshape(B * NH * 8, LPROJ)

    out = _main(g, ql, a2, qg, msk2d,
                kmat, vmat, wv8, W_go, row(b_go),
                row(ln_g[:HID]), row(ln_b[:HID]),
                row(ln_g[HID:]), row(ln_b[HID:]),
                W_m1[:HID], W_m1[HID:], row(b_m1), W_m2, row(b_m2))
    return out[:, :n, :].astype(x_flat.dtype)
